# Initial kernel scaffold; baseline (speedup 1.0000x reference)
import jax
import jax.numpy as jnp
from jax.experimental import pallas as pl


def kernel(x, edge_index, edge_attr, batch, gin_W1, gin_b1, gin_bn_g, gin_bn_b, gin_W2, gin_b2, gin_eps, bn_g, bn_b, vn_W1, vn_b1, vn_bn_g, vn_bn_b, vn_W2, vn_b2, vn_init, pred_W, pred_b):
    raise NotImplementedError("write your pallas kernel here")


# speedup vs baseline: 1.5269x; 1.5269x over previous
import functools

import jax
import jax.numpy as jnp
from jax import lax
from jax.experimental import pallas as pl
from jax.experimental.pallas import tpu as pltpu
from jax.experimental.pallas import tpu_sc as plsc

LAYERS = 5
EMB = 128
NNODE = 10000
NEDGE = 320000
NGRAPH = 100
NTASK = 128
GP = 128


def _DOT(a, b):
    return jnp.dot(a.astype(jnp.bfloat16), b.astype(jnp.bfloat16),
                   preferred_element_type=jnp.float32)


_CH = 128
_NBLK = NEDGE // _CH
_BLK_PER_CORE = _NBLK // 2
_BLK_BASE = _BLK_PER_CORE // 16
_BLK_EXTRA = _BLK_PER_CORE % 16
_NPAIR = _BLK_BASE // 2
_IDXROWS = _BLK_BASE + 1


def _msg_body(hl_hbm, src_hbm, ea_hbm, out_hbm,
              sidx_all, rows0, ea0, rows1, ea1, isem, gsem, esem, wsem0, wsem1):
    cid = lax.axis_index("c")
    sid = lax.axis_index("s")

    start_blk = cid * _BLK_PER_CORE + sid * _BLK_BASE + jnp.minimum(sid, _BLK_EXTRA)

    pltpu.async_copy(src_hbm.at[pl.ds(start_blk * _CH, _IDXROWS * _CH)],
                     sidx_all, isem).wait()

    def compute(rows, ea):
        @plsc.parallel_loop(0, _CH, unroll=2)
        def _edge_elem(e):
            for j in range(EMB // 16):
                sl = pl.ds(j * 16, 16)
                ea[e, sl] = jnp.maximum(rows[e, sl] + ea[e, sl], 0.0)

    def half(p, k, rows, ea, wsem):
        sidx = sidx_all.at[pl.ds(k * _CH, _CH)]
        gcp = pltpu.async_copy(hl_hbm.at[sidx], rows, gsem)
        esl = pl.ds((start_blk + k) * _CH, _CH)

        @pl.when(p > 0)
        def _drain_prev_write():
            pltpu.make_async_copy(ea, out_hbm.at[esl], wsem).wait()

        ecp = pltpu.async_copy(ea_hbm.at[esl], ea, esem)
        gcp.wait()
        ecp.wait()
        compute(rows, ea)
        pltpu.async_copy(ea, out_hbm.at[esl], wsem)

    def pair_body(p, carry):
        half(p, 2 * p, rows0, ea0, wsem0)
        half(p, 2 * p + 1, rows1, ea1, wsem1)
        return carry

    lax.fori_loop(0, _NPAIR, pair_body, 0)

    dummy = pl.ds(start_blk * _CH, _CH)
    pltpu.make_async_copy(ea0, out_hbm.at[dummy], wsem0).wait()
    pltpu.make_async_copy(ea1, out_hbm.at[dummy], wsem1).wait()

    @pl.when(sid < _BLK_EXTRA)
    def _tail():
        k = _BLK_BASE
        sidx = sidx_all.at[pl.ds(k * _CH, _CH)]
        esl = pl.ds((start_blk + k) * _CH, _CH)
        pltpu.async_copy(hl_hbm.at[sidx], rows0, gsem).wait()
        pltpu.async_copy(ea_hbm.at[esl], ea0, esem).wait()
        compute(rows0, ea0)
        pltpu.async_copy(ea0, out_hbm.at[esl], wsem0).wait()


@functools.cache
def _msg_kernel():
    return functools.partial(
        pl.kernel,
        mesh=plsc.VectorSubcoreMesh(core_axis_name="c", subcore_axis_name="s"),
        out_type=jax.ShapeDtypeStruct((NEDGE, EMB), jnp.float32),
        scratch_types=[
            pltpu.VMEM((_IDXROWS * _CH,), jnp.int32),
            pltpu.VMEM((_CH, EMB), jnp.float32),
            pltpu.VMEM((_CH, EMB), jnp.float32),
            pltpu.VMEM((_CH, EMB), jnp.float32),
            pltpu.VMEM((_CH, EMB), jnp.float32),
            pltpu.SemaphoreType.DMA,
            pltpu.SemaphoreType.DMA,
            pltpu.SemaphoreType.DMA,
            pltpu.SemaphoreType.DMA,
            pltpu.SemaphoreType.DMA,
        ],
    )(_msg_body)


def _msg_pass(hl, src_p, edge_attr):
    return _msg_kernel()(hl, src_p, edge_attr)


def _bn(h, g, b):
    mu = jnp.mean(h, axis=0)
    var = jnp.var(h, axis=0)
    return (h - mu) / jnp.sqrt(var + 1e-5) * g + b


def _init_body(x_ref, vn0_ref, hl_ref):
    hl_ref[...] = x_ref[...] + vn0_ref[...]


def _gin1_body(hl_ref, agg_ref, eps_ref, w1_ref, b1_ref, u_ref):
    z = (1.0 + eps_ref[0, 0]) * hl_ref[...] + agg_ref[...]
    u_ref[...] = _DOT(z, w1_ref[...]) + b1_ref[...]


def _gin2_body(u_ref, mu_ref, var_ref, g_ref, b_ref, w2_ref, b2_ref, w_ref):
    un = (u_ref[...] - mu_ref[...]) / jnp.sqrt(var_ref[...] + 1e-5) \
        * g_ref[...] + b_ref[...]
    un = jnp.maximum(un, 0.0)
    w_ref[...] = _DOT(un, w2_ref[...]) + b2_ref[...]


def _hl_next_body(w_ref, mu_ref, var_ref, g_ref, b_ref, ptn_ref, vnp_ref,
                  hl_ref):
    h = (w_ref[...] - mu_ref[...]) / jnp.sqrt(var_ref[...] + 1e-5) \
        * g_ref[...] + b_ref[...]
    h = jnp.maximum(h, 0.0)
    bcast = lax.dot_general(
        ptn_ref[...], vnp_ref[...], dimension_numbers=(((0,), (0,)), ((), ())),
        preferred_element_type=jnp.float32, precision=lax.Precision.HIGHEST)
    hl_ref[...] = h + bcast


def _hnode_body(w_ref, mu_ref, var_ref, g_ref, b_ref, h_ref):
    h_ref[...] = (w_ref[...] - mu_ref[...]) / jnp.sqrt(var_ref[...] + 1e-5) \
        * g_ref[...] + b_ref[...]


def _vn1_body(pooled_ref, vn_ref, w1_ref, b1_ref, t_ref):
    t = pooled_ref[...] + vn_ref[...]
    t_ref[...] = _DOT(t, w1_ref[...]) + b1_ref[...]


def _vn2_body(t_ref, mu_ref, var_ref, g_ref, b_ref, w2_ref, b2_ref, vn_ref):
    tn = (t_ref[...] - mu_ref[...]) / jnp.sqrt(var_ref[...] + 1e-5) \
        * g_ref[...] + b_ref[...]
    tn = jnp.maximum(tn, 0.0)
    vn_ref[...] = jnp.maximum(_DOT(tn, w2_ref[...]) + b2_ref[...], 0.0)


def _pred_body(hg_ref, pw_ref, pb_ref, out_ref):
    out_ref[...] = _DOT(hg_ref[...], pw_ref[...]) + pb_ref[...]


def _tc_call(body, out_shapes, *args):
    return pl.pallas_call(body, out_shape=out_shapes)(*args)


def kernel(x, edge_index, edge_attr, batch, gin_W1, gin_b1, gin_bn_g, gin_bn_b,
           gin_W2, gin_b2, gin_eps, bn_g, bn_b, vn_W1, vn_b1, vn_bn_g, vn_bn_b,
           vn_W2, vn_b2, vn_init, pred_W, pred_b):
    src = edge_index[0]
    dst = edge_index[1]
    src_p = jnp.concatenate([src, jnp.zeros((_CH,), jnp.int32)])
    ptn = (batch[None, :] == jnp.arange(GP, dtype=jnp.int32)[:, None]
           ).astype(jnp.float32)

    f32 = jnp.float32
    sds = jax.ShapeDtypeStruct
    row = lambda a: a[None, :]

    vn = jnp.broadcast_to(vn_init[None, :], (NGRAPH, EMB)).astype(f32)
    hl = _tc_call(_init_body, sds((NNODE, EMB), f32), x, row(vn_init))

    out = None
    for layer in range(LAYERS):
        msg = _msg_pass(hl, src_p, edge_attr)
        agg = jnp.zeros((NNODE, EMB), f32).at[dst].add(msg)
        if layer < LAYERS - 1:
            pooled = jnp.zeros((NGRAPH, EMB), f32).at[batch].add(hl)

        z = (1.0 + gin_eps[layer]) * hl + agg
        z = _DOT(z, gin_W1[layer]) + gin_b1[layer]
        z = _bn(z, gin_bn_g[layer], gin_bn_b[layer])
        z = jnp.maximum(z, 0.0)
        z = _DOT(z, gin_W2[layer]) + gin_b2[layer]
        z = _bn(z, bn_g[layer], bn_b[layer])
        if layer < LAYERS - 1:
            z = jnp.maximum(z, 0.0)

        if layer < LAYERS - 1:
            t = _tc_call(_vn1_body, sds((NGRAPH, 2 * EMB), f32),
                         pooled, vn, vn_W1[layer], row(vn_b1[layer]))
            mv, vv = jnp.mean(t, axis=0), jnp.var(t, axis=0)
            vn = _tc_call(_vn2_body, sds((NGRAPH, EMB), f32),
                          t, row(mv), row(vv), row(vn_bn_g[layer]),
                          row(vn_bn_b[layer]), vn_W2[layer], row(vn_b2[layer]))
            hl = z + vn[batch]
        else:
            counts = jnp.zeros((NGRAPH,), f32).at[batch].add(1.0)
            summed = jnp.zeros((NGRAPH, EMB), f32).at[batch].add(z)
            h_graph = summed / jnp.maximum(counts, 1.0)[:, None]
            out = _tc_call(_pred_body, sds((NGRAPH, NTASK), f32),
                           h_graph, pred_W, row(pred_b))
    return out
